# baseline (speedup 1.0000x reference)
import functools

import jax
import jax.numpy as jnp
from jax import lax
from jax.experimental import pallas as pl
from jax.experimental.pallas import tpu as pltpu
from jax.experimental.pallas import tpu_sc as plsc

NUM_CLASSES = 32
TABLE = NUM_CLASSES * NUM_CLASSES
LANES = 16


@functools.cache
def _make_lookup(n_rows: int, n_cols: int, rows_per_chunk: int, unroll: int):
    info = plsc.get_sparse_core_info()
    nc, ns = info.num_cores, info.num_subcores
    nw = nc * ns
    assert n_cols % (nw * 128) == 0
    cols_per_worker = n_cols // nw
    assert n_rows % rows_per_chunk == 0 and rows_per_chunk % 8 == 0
    n_chunks = n_rows // rows_per_chunk
    assert n_chunks >= 2

    mesh = plsc.VectorSubcoreMesh(core_axis_name="c", subcore_axis_name="s")

    @functools.partial(
        pl.kernel,
        out_type=jax.ShapeDtypeStruct((n_rows, n_cols), jnp.float32),
        mesh=mesh,
        compiler_params=pltpu.CompilerParams(
            needs_layout_passes=False, use_tc_tiling_on_sc=True),
        scratch_types=[
            pltpu.VMEM((NUM_CLASSES, NUM_CLASSES), jnp.float32),
            pltpu.VMEM((TABLE,), jnp.float32),
            pltpu.VMEM((rows_per_chunk, cols_per_worker), jnp.int32),
            pltpu.VMEM((rows_per_chunk, cols_per_worker), jnp.int32),
            pltpu.VMEM((rows_per_chunk, cols_per_worker), jnp.int32),
            pltpu.VMEM((rows_per_chunk, cols_per_worker), jnp.int32),
            pltpu.VMEM((rows_per_chunk, cols_per_worker), jnp.float32),
            pltpu.VMEM((rows_per_chunk, cols_per_worker), jnp.float32),
            pltpu.SemaphoreType.DMA,
            pltpu.SemaphoreType.DMA,
            pltpu.SemaphoreType.DMA,
            pltpu.SemaphoreType.DMA,
            pltpu.SemaphoreType.DMA,
            pltpu.SemaphoreType.DMA,
        ],
    )
    def lookup(ci_hbm, cj_hbm, lg_hbm, out_hbm,
               lg_v, tab_v, i0, i1, j0, j1, o0, o1,
               si0, si1, sj0, sj1, so0, so1):
        wid = lax.axis_index("s") * nc + lax.axis_index("c")
        w_col = wid * cols_per_worker
        ibufs, jbufs, obufs = (i0, i1), (j0, j1), (o0, o1)
        isems, jsems, osems = (si0, si1), (sj0, sj1), (so0, so1)

        def start_in(c):
            s = c % 2
            sl = (pl.ds(c * rows_per_chunk, rows_per_chunk),
                  pl.ds(w_col, cols_per_worker))
            di = pltpu.async_copy(ci_hbm.at[sl], ibufs[s], isems[s])
            dj = pltpu.async_copy(cj_hbm.at[sl], jbufs[s], jsems[s])
            return di, dj

        in_descs = {0: start_in(0), 1: start_in(1)}

        pltpu.sync_copy(lg_hbm, lg_v)

        @plsc.parallel_loop(0, TABLE, LANES)
        def build(base):
            p = lax.iota(jnp.int32, LANES) + base
            r = p >> 5
            c = p & (NUM_CLASSES - 1)
            a = plsc.load_gather(lg_v, [r, c])
            b = plsc.load_gather(lg_v, [c, r])
            x = (a + b) * 0.5
            tab_v[pl.ds(base, LANES)] = 1.0 / (1.0 + jnp.exp(-x))

        vecs_per_row = cols_per_worker // LANES
        n_vecs = rows_per_chunk * vecs_per_row
        out_descs = {}
        for c in range(n_chunks):
            s = c % 2
            di, dj = in_descs[c]
            di.wait()
            dj.wait()
            if c >= 2:
                out_descs[c - 2].wait()
            ib, jb, ob = ibufs[s], jbufs[s], obufs[s]

            def gath(k, ib=ib, jb=jb, ob=ob):
                r = k // vecs_per_row
                off = (k % vecs_per_row) * LANES
                iv = ib[r, pl.ds(off, LANES)]
                jv = jb[r, pl.ds(off, LANES)]
                idx = iv * NUM_CLASSES + jv
                ob[r, pl.ds(off, LANES)] = plsc.load_gather(tab_v, [idx])

            plsc.parallel_loop(0, n_vecs, 1, unroll=unroll)(gath)

            out_descs[c] = pltpu.async_copy(
                ob,
                out_hbm.at[pl.ds(c * rows_per_chunk, rows_per_chunk),
                           pl.ds(w_col, cols_per_worker)],
                osems[s])
            if c + 2 < n_chunks:
                in_descs[c + 2] = start_in(c + 2)
        out_descs[n_chunks - 2].wait()
        out_descs[n_chunks - 1].wait()

    return lookup


def kernel(class_i, class_j, compat_logits):
    n_rows, n_cols = class_i.shape
    ci = class_i.astype(jnp.int32).T
    cj = class_j.astype(jnp.int32).T
    lg = compat_logits.astype(jnp.float32)
    out_t = _make_lookup(n_cols, n_rows, 40, 8)(ci, cj, lg)
    return out_t.T


# speedup vs baseline: 1.0345x; 1.0345x over previous
import functools

import jax
import jax.numpy as jnp
from jax import lax
from jax.experimental import pallas as pl
from jax.experimental.pallas import tpu as pltpu
from jax.experimental.pallas import tpu_sc as plsc

NUM_CLASSES = 32
TABLE = NUM_CLASSES * NUM_CLASSES
LANES = 16


@functools.cache
def _make_lookup(n_rows: int, n_cols: int, rows_per_chunk: int, unroll: int):
    info = plsc.get_sparse_core_info()
    nc, ns = info.num_cores, info.num_subcores
    nw = nc * ns
    assert n_cols % (nw * 128) == 0
    cols_per_worker = n_cols // nw
    chunk_rows = [8, 32] + [rows_per_chunk] * ((n_rows - 80) // rows_per_chunk) + [32, 8]
    assert sum(chunk_rows) == n_rows
    assert all(r % 8 == 0 and r <= rows_per_chunk for r in chunk_rows)
    chunk_base = [sum(chunk_rows[:k]) for k in range(len(chunk_rows))]
    n_chunks = len(chunk_rows)
    assert n_chunks >= 2

    mesh = plsc.VectorSubcoreMesh(core_axis_name="c", subcore_axis_name="s")

    @functools.partial(
        pl.kernel,
        out_type=jax.ShapeDtypeStruct((n_rows, n_cols), jnp.float32),
        mesh=mesh,
        compiler_params=pltpu.CompilerParams(
            needs_layout_passes=False, use_tc_tiling_on_sc=True),
        scratch_types=[
            pltpu.VMEM((NUM_CLASSES, NUM_CLASSES), jnp.float32),
            pltpu.VMEM((TABLE,), jnp.float32),
            pltpu.VMEM((rows_per_chunk, cols_per_worker), jnp.int32),
            pltpu.VMEM((rows_per_chunk, cols_per_worker), jnp.int32),
            pltpu.VMEM((rows_per_chunk, cols_per_worker), jnp.int32),
            pltpu.VMEM((rows_per_chunk, cols_per_worker), jnp.int32),
            pltpu.VMEM((rows_per_chunk, cols_per_worker), jnp.float32),
            pltpu.VMEM((rows_per_chunk, cols_per_worker), jnp.float32),
            pltpu.SemaphoreType.DMA,
            pltpu.SemaphoreType.DMA,
            pltpu.SemaphoreType.DMA,
            pltpu.SemaphoreType.DMA,
            pltpu.SemaphoreType.DMA,
            pltpu.SemaphoreType.DMA,
        ],
    )
    def lookup(ci_hbm, cj_hbm, lg_hbm, out_hbm,
               lg_v, tab_v, i0, i1, j0, j1, o0, o1,
               si0, si1, sj0, sj1, so0, so1):
        wid = lax.axis_index("s") * nc + lax.axis_index("c")
        w_col = wid * cols_per_worker
        ibufs, jbufs, obufs = (i0, i1), (j0, j1), (o0, o1)
        isems, jsems, osems = (si0, si1), (sj0, sj1), (so0, so1)

        def start_in(c):
            s = c % 2
            rows = chunk_rows[c]
            sl = (pl.ds(chunk_base[c], rows), pl.ds(w_col, cols_per_worker))
            bsl = pl.ds(0, rows)
            di = pltpu.async_copy(ci_hbm.at[sl], ibufs[s].at[bsl], isems[s])
            dj = pltpu.async_copy(cj_hbm.at[sl], jbufs[s].at[bsl], jsems[s])
            return di, dj

        in_descs = {0: start_in(0), 1: start_in(1)}

        pltpu.sync_copy(lg_hbm, lg_v)

        @plsc.parallel_loop(0, TABLE, LANES)
        def build(base):
            p = lax.iota(jnp.int32, LANES) + base
            r = p >> 5
            c = p & (NUM_CLASSES - 1)
            a = plsc.load_gather(lg_v, [r, c])
            b = plsc.load_gather(lg_v, [c, r])
            x = (a + b) * 0.5
            tab_v[pl.ds(base, LANES)] = 1.0 / (1.0 + jnp.exp(-x))

        vecs_per_row = cols_per_worker // LANES
        out_descs = {}
        for c in range(n_chunks):
            s = c % 2
            di, dj = in_descs[c]
            di.wait()
            dj.wait()
            if c >= 2:
                out_descs[c - 2].wait()
            ib, jb, ob = ibufs[s], jbufs[s], obufs[s]

            def gath(k, ib=ib, jb=jb, ob=ob):
                r = k // vecs_per_row
                off = (k % vecs_per_row) * LANES
                iv = ib[r, pl.ds(off, LANES)]
                jv = jb[r, pl.ds(off, LANES)]
                idx = iv * NUM_CLASSES + jv
                ob[r, pl.ds(off, LANES)] = plsc.load_gather(tab_v, [idx])

            n_vecs = chunk_rows[c] * vecs_per_row
            plsc.parallel_loop(0, n_vecs, 1, unroll=unroll)(gath)

            out_descs[c] = pltpu.async_copy(
                ob.at[pl.ds(0, chunk_rows[c])],
                out_hbm.at[pl.ds(chunk_base[c], chunk_rows[c]),
                           pl.ds(w_col, cols_per_worker)],
                osems[s])
            if c + 2 < n_chunks:
                in_descs[c + 2] = start_in(c + 2)
        out_descs[n_chunks - 2].wait()
        out_descs[n_chunks - 1].wait()

    return lookup


def kernel(class_i, class_j, compat_logits):
    n_rows, n_cols = class_i.shape
    ci = class_i.astype(jnp.int32).T
    cj = class_j.astype(jnp.int32).T
    lg = compat_logits.astype(jnp.float32)
    out_t = _make_lookup(n_cols, n_rows, 40, 4)(ci, cj, lg)
    return out_t.T
